# Initial kernel scaffold; baseline (speedup 1.0000x reference)
import jax
import jax.numpy as jnp
from jax.experimental import pallas as pl


def kernel(x, gate_up_proj, down_proj, gate_up_proj_bias, down_proj_bias):
    raise NotImplementedError("write your pallas kernel here")


# speedup vs baseline: 1.2706x; 1.2706x over previous
import functools

import jax
import jax.numpy as jnp
from jax.experimental import pallas as pl


def _ffn_kernel(x_ref, gw_ref, uw_ref, gb_ref, ub_ref, dw_ref, db_ref, o_ref):
    ti = pl.program_id(2)
    x = x_ref[0]
    g = jnp.dot(x, gw_ref[0], preferred_element_type=jnp.float32) + gb_ref[0]
    u = jnp.dot(x, uw_ref[0], preferred_element_type=jnp.float32) + ub_ref[0]
    h = (g * jax.nn.sigmoid(g)) * u
    acc = jnp.dot(h, dw_ref[0], preferred_element_type=jnp.float32)

    @pl.when(ti == 0)
    def _init():
        o_ref[0] = acc + db_ref[0]

    @pl.when(ti != 0)
    def _accum():
        o_ref[0] += acc


@functools.partial(jax.jit, static_argnames=("tm", "ti"))
def _ffn(x, gate_up_proj, down_proj, gate_up_proj_bias, down_proj_bias,
         tm: int, ti: int):
    e, t, h = x.shape
    i = down_proj.shape[1]
    n_ti = i // ti
    gub = gate_up_proj_bias.reshape(e, 1, 2 * i)
    db = down_proj_bias.reshape(e, 1, h)

    grid = (e, t // tm, n_ti)
    return pl.pallas_call(
        _ffn_kernel,
        grid=grid,
        in_specs=[
            pl.BlockSpec((1, tm, h), lambda ei, tmi, tii: (ei, tmi, 0)),
            pl.BlockSpec((1, h, ti), lambda ei, tmi, tii: (ei, 0, tii)),
            pl.BlockSpec((1, h, ti),
                         lambda ei, tmi, tii, n=n_ti: (ei, 0, tii + n)),
            pl.BlockSpec((1, 1, ti), lambda ei, tmi, tii: (ei, 0, tii)),
            pl.BlockSpec((1, 1, ti),
                         lambda ei, tmi, tii, n=n_ti: (ei, 0, tii + n)),
            pl.BlockSpec((1, ti, h), lambda ei, tmi, tii: (ei, tii, 0)),
            pl.BlockSpec((1, 1, h), lambda ei, tmi, tii: (ei, 0, 0)),
        ],
        out_specs=pl.BlockSpec((1, tm, h), lambda ei, tmi, tii: (ei, tmi, 0)),
        out_shape=jax.ShapeDtypeStruct((e, t, h), jnp.float32),
    )(x, gate_up_proj, gate_up_proj, gub, gub, down_proj, db)


def kernel(x, gate_up_proj, down_proj, gate_up_proj_bias, down_proj_bias):
    t = x.shape[1]
    i = down_proj.shape[1]
    tm = min(t, 1024)
    ti = min(i, 512)
    return _ffn(x, gate_up_proj, down_proj, gate_up_proj_bias, down_proj_bias,
                tm, ti)


# speedup vs baseline: 1.3488x; 1.0616x over previous
import functools

import jax
import jax.numpy as jnp
from jax.experimental import pallas as pl


def _ffn_kernel(x_ref, gw_ref, uw_ref, gb_ref, ub_ref, dw_ref, db_ref, o_ref):
    ti = pl.program_id(2)
    x = x_ref[0]
    g = jnp.dot(x, gw_ref[0], preferred_element_type=jnp.float32) + gb_ref[0]
    u = jnp.dot(x, uw_ref[0], preferred_element_type=jnp.float32) + ub_ref[0]
    h = (g * jax.nn.sigmoid(g)) * u
    acc = jnp.dot(h, dw_ref[0], preferred_element_type=jnp.float32)

    @pl.when(ti == 0)
    def _init():
        o_ref[0] = acc + db_ref[0]

    @pl.when(ti != 0)
    def _accum():
        o_ref[0] += acc


@functools.partial(jax.jit, static_argnames=("tm", "ti"))
def _ffn(x, gate_up_proj, down_proj, gate_up_proj_bias, down_proj_bias,
         tm: int, ti: int):
    e, t, h = x.shape
    i = down_proj.shape[1]
    n_ti = i // ti
    gub = gate_up_proj_bias.reshape(e, 1, 2 * i)
    db = down_proj_bias.reshape(e, 1, h)

    grid = (e, t // tm, n_ti)
    return pl.pallas_call(
        _ffn_kernel,
        grid=grid,
        in_specs=[
            pl.BlockSpec((1, tm, h), lambda ei, tmi, tii: (ei, tmi, 0)),
            pl.BlockSpec((1, h, ti), lambda ei, tmi, tii: (ei, 0, tii)),
            pl.BlockSpec((1, h, ti),
                         lambda ei, tmi, tii, n=n_ti: (ei, 0, tii + n)),
            pl.BlockSpec((1, 1, ti), lambda ei, tmi, tii: (ei, 0, tii)),
            pl.BlockSpec((1, 1, ti),
                         lambda ei, tmi, tii, n=n_ti: (ei, 0, tii + n)),
            pl.BlockSpec((1, ti, h), lambda ei, tmi, tii: (ei, tii, 0)),
            pl.BlockSpec((1, 1, h), lambda ei, tmi, tii: (ei, 0, 0)),
        ],
        out_specs=pl.BlockSpec((1, tm, h), lambda ei, tmi, tii: (ei, tmi, 0)),
        out_shape=jax.ShapeDtypeStruct((e, t, h), jnp.float32),
    )(x, gate_up_proj, gate_up_proj, gub, gub, down_proj, db)


def kernel(x, gate_up_proj, down_proj, gate_up_proj_bias, down_proj_bias):
    t = x.shape[1]
    i = down_proj.shape[1]
    tm = min(t, 1024)
    ti = min(i, 1024)
    return _ffn(x, gate_up_proj, down_proj, gate_up_proj_bias, down_proj_bias,
                tm, ti)


# speedup vs baseline: 1.3749x; 1.0193x over previous
import functools

import jax
import jax.numpy as jnp
from jax.experimental import pallas as pl


def _ffn_kernel(x_ref, gw_ref, uw_ref, gb_ref, ub_ref, dw_ref, db_ref, o_ref):
    ti = pl.program_id(2)
    x = x_ref[0].astype(jnp.bfloat16)
    g = jnp.dot(x, gw_ref[0].astype(jnp.bfloat16),
                preferred_element_type=jnp.float32) + gb_ref[0]
    u = jnp.dot(x, uw_ref[0].astype(jnp.bfloat16),
                preferred_element_type=jnp.float32) + ub_ref[0]
    h = (g * jax.nn.sigmoid(g)) * u
    acc = jnp.dot(h.astype(jnp.bfloat16), dw_ref[0].astype(jnp.bfloat16),
                  preferred_element_type=jnp.float32)

    @pl.when(ti == 0)
    def _init():
        o_ref[0] = acc + db_ref[0]

    @pl.when(ti != 0)
    def _accum():
        o_ref[0] += acc


@functools.partial(jax.jit, static_argnames=("tm", "ti"))
def _ffn(x, gate_up_proj, down_proj, gate_up_proj_bias, down_proj_bias,
         tm: int, ti: int):
    e, t, h = x.shape
    i = down_proj.shape[1]
    n_ti = i // ti
    gub = gate_up_proj_bias.reshape(e, 1, 2 * i)
    db = down_proj_bias.reshape(e, 1, h)

    grid = (e, t // tm, n_ti)
    return pl.pallas_call(
        _ffn_kernel,
        grid=grid,
        in_specs=[
            pl.BlockSpec((1, tm, h), lambda ei, tmi, tii: (ei, tmi, 0)),
            pl.BlockSpec((1, h, ti), lambda ei, tmi, tii: (ei, 0, tii)),
            pl.BlockSpec((1, h, ti),
                         lambda ei, tmi, tii, n=n_ti: (ei, 0, tii + n)),
            pl.BlockSpec((1, 1, ti), lambda ei, tmi, tii: (ei, 0, tii)),
            pl.BlockSpec((1, 1, ti),
                         lambda ei, tmi, tii, n=n_ti: (ei, 0, tii + n)),
            pl.BlockSpec((1, ti, h), lambda ei, tmi, tii: (ei, tii, 0)),
            pl.BlockSpec((1, 1, h), lambda ei, tmi, tii: (ei, 0, 0)),
        ],
        out_specs=pl.BlockSpec((1, tm, h), lambda ei, tmi, tii: (ei, tmi, 0)),
        out_shape=jax.ShapeDtypeStruct((e, t, h), jnp.float32),
    )(x, gate_up_proj, gate_up_proj, gub, gub, down_proj, db)


def kernel(x, gate_up_proj, down_proj, gate_up_proj_bias, down_proj_bias):
    t = x.shape[1]
    i = down_proj.shape[1]
    tm = min(t, 1024)
    ti = min(i, 1024)
    return _ffn(x, gate_up_proj, down_proj, gate_up_proj_bias, down_proj_bias,
                tm, ti)
